# baseline (speedup 1.0000x reference)
import jax
import jax.numpy as jnp
from jax import lax
from jax.experimental import pallas as pl
from jax.experimental.pallas import tpu as pltpu
from jax.experimental.pallas import tpu_sc as plsc

N_CAT = 26
N_NUM = 13
D = 32
B = 16384
CAT_SIZE = 100000

NC = 2
NS = 16
NW = NC * NS
BPW = B // NW
IDX_CHUNK = 128
SPF = BPW // IDX_CHUNK
L = 16


def _body(xnum_hbm, xcat_hbm, nw_hbm, nb_hbm, tbl_hbm, cb_hbm, out_hbm,
          xcat_v, xnum_v, gbuf0, gbuf1, obuf0, obuf1, nw_v, nb_v, cb_v,
          gsem0, gsem1, osem0, osem1):
    wid = lax.axis_index("c") * NS + lax.axis_index("s")
    b0 = wid * BPW

    pltpu.sync_copy(xcat_hbm.at[:, pl.ds(b0, BPW)], xcat_v)
    pltpu.sync_copy(xnum_hbm.at[:, pl.ds(b0, BPW)], xnum_v)
    pltpu.sync_copy(nw_hbm, nw_v)
    pltpu.sync_copy(nb_hbm, nb_v)
    pltpu.sync_copy(cb_hbm, cb_v)

    gbuf = (gbuf0, gbuf1)
    obuf = (obuf0, obuf1)
    gsems = (gsem0, gsem1)
    osems = (osem0, osem1)

    def fire_gather(f):
        descs = []
        for c in range(SPF):
            descs.append(pltpu.async_copy(
                tbl_hbm.at[xcat_v.at[f, pl.ds(c * IDX_CHUNK, IDX_CHUNK)]],
                gbuf[f % 2].at[pl.ds(c * IDX_CHUNK, IDX_CHUNK), :],
                gsems[f % 2]))
        return descs

    g_descs = {0: fire_gather(0)}

    @pl.loop(BPW // L, N_CAT * (BPW // L), unroll=8)
    def _offset_add(i):
        f = i // (BPW // L)
        k = i % (BPW // L)
        sl = pl.ds(k * L, L)
        xcat_v[f, sl] = xcat_v[f, sl] + f * CAT_SIZE

    out_descs = {}
    for f in range(N_CAT):
        buf = f % 2
        if f + 1 < N_CAT:
            g_descs[f + 1] = fire_gather(f + 1)
        for d in g_descs[f]:
            d.wait()
        if f - 2 >= 0:
            out_descs[f - 2].wait()
        g = gbuf[buf]
        o = obuf[buf]
        cb_lo = cb_v[f, pl.ds(0, L)]
        cb_hi = cb_v[f, pl.ds(L, L)]

        @pl.loop(0, BPW, unroll=8)
        def _bias_add(i, g=g, o=o, cb_lo=cb_lo, cb_hi=cb_hi):
            o[i, 0, pl.ds(0, L)] = g[i, pl.ds(0, L)] + cb_lo
            o[i, 0, pl.ds(L, L)] = g[i, pl.ds(L, L)] + cb_hi

        out_descs[f] = pltpu.async_copy(
            o, out_hbm.at[pl.ds(b0, BPW), pl.ds(N_NUM + f, 1), :],
            osems[buf])

    num_descs = {}
    for j in range(N_NUM):
        buf = j % 2
        if j < 2:
            out_descs[N_CAT - 2 + j].wait()
        else:
            num_descs[j - 2].wait()
        o = obuf[buf]
        w_lo = nw_v[j, pl.ds(0, L)]
        w_hi = nw_v[j, pl.ds(L, L)]
        a_lo = nb_v[j, pl.ds(0, L)]
        a_hi = nb_v[j, pl.ds(L, L)]

        @pl.loop(0, BPW // L)
        def _num_emb(t, j=j, o=o, w_lo=w_lo, w_hi=w_hi, a_lo=a_lo, a_hi=a_hi):
            xv = xnum_v[j, pl.ds(t * L, L)]
            for e in range(L):
                xs = xv[e]
                o[t * L + e, 0, pl.ds(0, L)] = w_lo * xs + a_lo
                o[t * L + e, 0, pl.ds(L, L)] = w_hi * xs + a_hi

        num_descs[j] = pltpu.async_copy(
            o, out_hbm.at[pl.ds(b0, BPW), pl.ds(j, 1), :], osems[buf])

    num_descs[N_NUM - 2].wait()
    num_descs[N_NUM - 1].wait()


_sc_tokenize = pl.kernel(
    _body,
    out_type=jax.ShapeDtypeStruct((B, N_NUM + N_CAT, D), jnp.float32),
    mesh=plsc.VectorSubcoreMesh(core_axis_name="c", subcore_axis_name="s",
                                num_cores=NC, num_subcores=NS),
    compiler_params=pltpu.CompilerParams(use_tc_tiling_on_sc=False),
    scratch_types=[
        pltpu.VMEM((N_CAT, BPW), jnp.int32),
        pltpu.VMEM((N_NUM, BPW), jnp.float32),
        pltpu.VMEM((BPW, D), jnp.float32),
        pltpu.VMEM((BPW, D), jnp.float32),
        pltpu.VMEM((BPW, 1, D), jnp.float32),
        pltpu.VMEM((BPW, 1, D), jnp.float32),
        pltpu.VMEM((N_NUM, D), jnp.float32),
        pltpu.VMEM((N_NUM, D), jnp.float32),
        pltpu.VMEM((N_CAT, D), jnp.float32),
        pltpu.SemaphoreType.DMA,
        pltpu.SemaphoreType.DMA,
        pltpu.SemaphoreType.DMA,
        pltpu.SemaphoreType.DMA,
    ],
)


@jax.jit
def kernel(x_num, x_cat, num_weight, num_bias, cat_table, cat_bias):
    xnum_t = jnp.asarray(x_num, jnp.float32).T
    xcat_t = jnp.asarray(x_cat, jnp.int32).T
    return _sc_tokenize(xnum_t, xcat_t, num_weight, num_bias,
                        cat_table, cat_bias)
